# baseline (speedup 1.0000x reference)
import functools

import jax
import jax.numpy as jnp
from jax import lax
from jax.experimental import pallas as pl
from jax.experimental.pallas import tpu as pltpu
from jax.experimental.pallas import tpu_sc as plsc

BUCKETS = 100000
D = 16
B = 16384
NC = 2
NS = 16
NW = NC * NS
BPW = B // NW
CHUNK = 128
NCHUNK = BPW // CHUNK
GROUPS = BPW // D


@functools.partial(
    pl.kernel,
    out_type=jax.ShapeDtypeStruct((B,), jnp.float32),
    mesh=plsc.VectorSubcoreMesh(core_axis_name="c", subcore_axis_name="s"),
    compiler_params=pltpu.CompilerParams(
        needs_layout_passes=False, use_tc_tiling_on_sc=False),
    scratch_types=[
        pltpu.VMEM((NCHUNK, CHUNK), jnp.int32),
        pltpu.VMEM((NCHUNK, CHUNK), jnp.int32),
        pltpu.VMEM((BPW, D), jnp.float32),
        pltpu.VMEM((BPW, D), jnp.float32),
        pltpu.VMEM((BPW,), jnp.float32),
        pltpu.VMEM((D,), jnp.float32),
        pltpu.VMEM((D,), jnp.float32),
        pltpu.SemaphoreType.DMA,
    ],
)
def _fm_sc(uid_hbm, tid_hbm, utab_hbm, itab_hbm, w_hbm, b_hbm, out_hbm,
           idx_u, idx_t, yu, yt, out_v, w_v, b_v, sem):
    wid = lax.axis_index("s") * NC + lax.axis_index("c")
    base = wid * BPW

    pltpu.sync_copy(uid_hbm.at[wid], idx_u)
    pltpu.sync_copy(tid_hbm.at[wid], idx_t)
    pltpu.sync_copy(w_hbm, w_v)
    pltpu.sync_copy(b_hbm, b_v)

    copies = []
    for j in range(NCHUNK):
        dst = yu.at[pl.ds(j * CHUNK, CHUNK), :]
        copies.append(pltpu.async_copy(utab_hbm.at[idx_u.at[j]], dst, sem))
        dst = yt.at[pl.ds(j * CHUNK, CHUNK), :]
        copies.append(pltpu.async_copy(itab_hbm.at[idx_t.at[j]], dst, sem))
    for c in copies:
        c.wait()

    w = w_v[...]
    bb = b_v[...]
    iot = lax.iota(jnp.int32, D)

    def group(g, carry):
        rows = g * D + iot
        acc = jnp.zeros((D,), jnp.float32)
        for d in range(D):
            cols = jnp.full((D,), d, jnp.int32)
            u = plsc.load_gather(yu, [rows, cols])
            t = plsc.load_gather(yt, [rows, cols])
            acc = acc + u * t
        z = acc * w + bb
        out_v[pl.ds(g * D, D)] = 1.0 / (1.0 + jnp.exp(-z))
        return carry

    lax.fori_loop(0, GROUPS, group, 0)
    pltpu.sync_copy(out_v, out_hbm.at[pl.ds(base, BPW)])


def kernel(f_uid, f_tid, user_table, item_table, W, b):
    uid = f_uid.astype(jnp.int32).reshape(NW, NCHUNK, CHUNK)
    tid = f_tid.astype(jnp.int32).reshape(NW, NCHUNK, CHUNK)
    wvec = jnp.broadcast_to(W.astype(jnp.float32).reshape(()), (D,))
    bvec = jnp.broadcast_to(b.astype(jnp.float32).reshape(()), (D,))
    y = _fm_sc(uid, tid, user_table, item_table, wvec, bvec)
    return y.reshape(B, 1)
